# baseline (speedup 1.0000x reference)
import functools

import jax
import jax.numpy as jnp
import numpy as np
from jax import lax
from jax.experimental import pallas as pl
from jax.experimental.pallas import tpu as pltpu
from jax.experimental.pallas import tpu_sc as plsc

_RES = np.float32(0.1)
_MASK = np.int32(1048576 - 1)
_P0 = np.int32(73856093)
_P1 = np.int32(19349669)
_P2 = np.int32(83492791)
_L = 16
_NW = 32
_CHUNK = 256
_ISUB = 128


def _floor_res(q):
    t = q / _RES
    i = t.astype(jnp.int32)
    f = i.astype(jnp.float32)
    return jnp.where(f > t, i - np.int32(1), i)


@functools.lru_cache(maxsize=None)
def _make_kernel(n, d, zero_row):
    pw = n // _NW
    n_chunks = pw // _CHUNK
    mesh = plsc.VectorSubcoreMesh(core_axis_name="c", subcore_axis_name="s")

    def buf_set():
        return [
            pltpu.VMEM((_CHUNK * 3,), jnp.float32),
            pltpu.VMEM((_CHUNK,), jnp.int32),
            pltpu.VMEM((_CHUNK,), jnp.int32),
            pltpu.VMEM((_CHUNK,), jnp.int32),
            pltpu.VMEM((_CHUNK, d), jnp.float32),
            pltpu.SemaphoreType.DMA,
            pltpu.SemaphoreType.DMA,
            pltpu.SemaphoreType.DMA,
        ]

    @functools.partial(
        pl.kernel,
        mesh=mesh,
        compiler_params=pltpu.CompilerParams(
            use_tc_tiling_on_sc=False, needs_layout_passes=False),
        out_type=jax.ShapeDtypeStruct((n, d), jnp.float32),
        scratch_types=buf_set() + buf_set() + [pltpu.SemaphoreType.DMA],
    )
    def k(q_h, buf_h, feat_h, out_h,
          qA, hashA, voxA, idxA, rowsA, semqA, semrA, semoA,
          qB, hashB, voxB, idxB, rowsB, semqB, semrB, semoB,
          sem_vox):
        wid = lax.axis_index("s") * np.int32(2) + lax.axis_index("c")
        base = wid * np.int32(pw)
        it3 = lax.iota(jnp.int32, 16) * np.int32(3)
        A = (qA, hashA, voxA, idxA, rowsA, semqA, semrA, semoA)
        B = (qB, hashB, voxB, idxB, rowsB, semqB, semrB, semoB)

        def half(c, cur, prv):
            (q_c, hash_c, vox_c, idx_c, rows_c, semq_c, semr_c, semo_c) = cur
            (q_p, _, _, _, rows_p, semq_p, semr_p, semo_p) = prv
            off = pl.multiple_of(base + c * np.int32(_CHUNK), _CHUNK)
            off3 = pl.multiple_of(off * np.int32(3), 8)
            pltpu.make_async_copy(
                q_h.at[pl.ds(off3, _CHUNK * 3)], q_c, semq_c).wait()
            for g in range(_CHUNK // _L):
                wx = it3 + np.int32(g * 48)
                gx = _floor_res(plsc.load_gather(q_c, [wx]))
                gy = _floor_res(plsc.load_gather(q_c, [wx + np.int32(1)]))
                gz = _floor_res(plsc.load_gather(q_c, [wx + np.int32(2)]))
                hash_c[pl.ds(g * _L, _L)] = (
                    gx * _P0 + gy * _P1 + gz * _P2) & _MASK
            hs = [
                pltpu.async_copy(
                    buf_h.at[hash_c.at[pl.ds(s * _ISUB, _ISUB)]],
                    vox_c.at[pl.ds(s * _ISUB, _ISUB)], sem_vox)
                for s in range(_CHUNK // _ISUB)
            ]

            @pl.when(c < np.int32(n_chunks - 1))
            def _():
                noff3 = pl.multiple_of(off3 + np.int32(_CHUNK * 3), 8)
                pltpu.async_copy(
                    q_h.at[pl.ds(noff3, _CHUNK * 3)], q_p, semq_p)

            for h in hs:
                h.wait()
            for g in range(_CHUNK // _L):
                sl = pl.ds(g * _L, _L)
                v = vox_c[sl]
                idx_c[sl] = jnp.where(v >= np.int32(0), v, np.int32(zero_row))

            @pl.when(c >= np.int32(2))
            def _():
                pltpu.make_async_copy(
                    rows_c, out_h.at[pl.ds(0, _CHUNK)], semo_c).wait()

            def row_body(_, p16):
                v = idx_c[pl.ds(p16, _L)]
                for lane in range(_L):
                    woff = pl.multiple_of(v[lane] * np.int32(d), 8)
                    pltpu.async_copy(
                        feat_h.at[pl.ds(woff, d)],
                        rows_c.at[p16 + np.int32(lane)], semr_c)
                return p16 + np.int32(_L)

            lax.fori_loop(0, _CHUNK // _L, row_body, np.int32(0))

            @pl.when(c >= np.int32(1))
            def _():
                pltpu.make_async_copy(
                    out_h.at[pl.ds(0, _CHUNK)], rows_p, semr_p).wait()
                poff = pl.multiple_of(off - np.int32(_CHUNK), _CHUNK)
                pltpu.async_copy(
                    rows_p, out_h.at[pl.ds(poff, _CHUNK)], semo_p)

        base3 = pl.multiple_of(base * np.int32(3), 8)
        pltpu.async_copy(q_h.at[pl.ds(base3, _CHUNK * 3)], qA, semqA)

        def pair(i2, c):
            half(c, A, B)
            half(c + np.int32(1), B, A)
            return c + np.int32(2)

        lax.fori_loop(0, n_chunks // 2, pair, np.int32(0))

        pltpu.make_async_copy(out_h.at[pl.ds(0, _CHUNK)], rowsB, semrB).wait()
        off_last = pl.multiple_of(
            base + np.int32((n_chunks - 1) * _CHUNK), _CHUNK)
        pltpu.async_copy(rowsB, out_h.at[pl.ds(off_last, _CHUNK)], semoB)
        pltpu.make_async_copy(rowsA, out_h.at[pl.ds(0, _CHUNK)], semoA).wait()
        pltpu.make_async_copy(rowsB, out_h.at[pl.ds(0, _CHUNK)], semoB).wait()

    return k


def kernel(query_pts, features, buffer_voxel_index):
    n = query_pts.shape[0]
    nv, d = features.shape
    q_flat = query_pts.reshape(-1)
    buf = buffer_voxel_index.astype(jnp.int32)
    feat_ext = jnp.concatenate(
        [features.astype(jnp.float32).reshape(-1), jnp.zeros((d,), jnp.float32)])
    return _make_kernel(n, d, nv)(q_flat, buf, feat_ext)


# speedup vs baseline: 1.1939x; 1.1939x over previous
import functools

import jax
import jax.numpy as jnp
import numpy as np
from jax import lax
from jax.experimental import pallas as pl
from jax.experimental.pallas import tpu as pltpu
from jax.experimental.pallas import tpu_sc as plsc

_RES = np.float32(0.1)
_MASK = np.int32(1048576 - 1)
_P0 = np.int32(73856093)
_P1 = np.int32(19349669)
_P2 = np.int32(83492791)
_L = 16
_NW = 32
_CHUNK = 256
_NSTREAM = 128


def _floor_res(q):
    t = q / _RES
    i = t.astype(jnp.int32)
    f = i.astype(jnp.float32)
    return jnp.where(f > t, i - np.int32(1), i)


@functools.lru_cache(maxsize=None)
def _make_kernel(n, d, zero_row):
    pw = n // _NW
    n_chunks = pw // _CHUNK
    mesh = plsc.VectorSubcoreMesh(core_axis_name="c", subcore_axis_name="s")

    def buf_set():
        return [
            pltpu.VMEM((_CHUNK,), jnp.float32),
            pltpu.VMEM((_CHUNK,), jnp.float32),
            pltpu.VMEM((_CHUNK,), jnp.float32),
            pltpu.VMEM((_CHUNK,), jnp.int32),
            pltpu.VMEM((_CHUNK,), jnp.int32),
            pltpu.VMEM((_CHUNK,), jnp.int32),
            pltpu.VMEM((_CHUNK, d), jnp.float32),
            pltpu.SemaphoreType.DMA,
            pltpu.SemaphoreType.DMA,
            pltpu.SemaphoreType.DMA,
        ]

    @functools.partial(
        pl.kernel,
        mesh=mesh,
        compiler_params=pltpu.CompilerParams(use_tc_tiling_on_sc=False),
        out_type=jax.ShapeDtypeStruct((n, d), jnp.float32),
        scratch_types=buf_set() + buf_set() + [pltpu.SemaphoreType.DMA],
    )
    def k(qx_h, qy_h, qz_h, buf_h, feat_h, out_h,
          qxA, qyA, qzA, hashA, voxA, idxA, rowsA, semqA, semrA, semoA,
          qxB, qyB, qzB, hashB, voxB, idxB, rowsB, semqB, semrB, semoB,
          sem_vox):
        wid = lax.axis_index("s") * np.int32(2) + lax.axis_index("c")
        base = wid * np.int32(pw)
        A = (qxA, qyA, qzA, hashA, voxA, idxA, rowsA, semqA, semrA, semoA)
        B = (qxB, qyB, qzB, hashB, voxB, idxB, rowsB, semqB, semrB, semoB)

        def q_prefetch(off, bufs):
            qx_c, qy_c, qz_c = bufs[0], bufs[1], bufs[2]
            semq_c = bufs[7]
            pltpu.async_copy(qx_h.at[pl.ds(off, _CHUNK)], qx_c, semq_c)
            pltpu.async_copy(qy_h.at[pl.ds(off, _CHUNK)], qy_c, semq_c)
            pltpu.async_copy(qz_h.at[pl.ds(off, _CHUNK)], qz_c, semq_c)

        def half(c, cur, prv):
            (qx_c, qy_c, qz_c, hash_c, vox_c, idx_c, rows_c,
             semq_c, semr_c, semo_c) = cur
            rows_p, semr_p, semo_p = prv[6], prv[8], prv[9]
            off = pl.multiple_of(base + c * np.int32(_CHUNK), _CHUNK)
            for q_ref in (qx_c, qy_c, qz_c):
                pltpu.make_async_copy(
                    qx_h.at[pl.ds(off, _CHUNK)], q_ref, semq_c).wait()
            for g in range(_CHUNK // _L):
                sl = pl.ds(g * _L, _L)
                gx = _floor_res(qx_c[sl])
                gy = _floor_res(qy_c[sl])
                gz = _floor_res(qz_c[sl])
                hash_c[sl] = (gx * _P0 + gy * _P1 + gz * _P2) & _MASK
            hs = [
                pltpu.async_copy(
                    buf_h.at[hash_c.at[pl.ds(s * 128, 128)]],
                    vox_c.at[pl.ds(s * 128, 128)], sem_vox)
                for s in range(_CHUNK // 128)
            ]

            @pl.when(c < np.int32(n_chunks - 1))
            def _():
                q_prefetch(off + np.int32(_CHUNK), prv)

            for h in hs:
                h.wait()
            for g in range(_CHUNK // _L):
                sl = pl.ds(g * _L, _L)
                v = vox_c[sl]
                idx_c[sl] = jnp.where(v >= np.int32(0), v, np.int32(zero_row))

            @pl.when(c >= np.int32(2))
            def _():
                pltpu.make_async_copy(
                    rows_c, out_h.at[pl.ds(0, _CHUNK)], semo_c).wait()

            pltpu.async_copy(
                feat_h.at[idx_c.at[pl.ds(0, _NSTREAM)]],
                rows_c.at[pl.ds(0, _NSTREAM)], semr_c)

            def row_body(_, p16):
                v = idx_c[pl.ds(p16, _L)]
                for lane in range(_L):
                    pltpu.async_copy(
                        feat_h.at[v[lane]],
                        rows_c.at[p16 + np.int32(lane)], semr_c)
                return p16 + np.int32(_L)

            lax.fori_loop(0, (_CHUNK - _NSTREAM) // _L, row_body,
                          np.int32(_NSTREAM))

            @pl.when(c >= np.int32(1))
            def _():
                pltpu.make_async_copy(
                    out_h.at[pl.ds(0, _CHUNK)], rows_p, semr_p).wait()
                poff = pl.multiple_of(off - np.int32(_CHUNK), _CHUNK)
                pltpu.async_copy(
                    rows_p, out_h.at[pl.ds(poff, _CHUNK)], semo_p)

        q_prefetch(pl.multiple_of(base, _CHUNK), A)

        def pair(i2, c):
            half(c, A, B)
            half(c + np.int32(1), B, A)
            return c + np.int32(2)

        lax.fori_loop(0, n_chunks // 2, pair, np.int32(0))

        pltpu.make_async_copy(out_h.at[pl.ds(0, _CHUNK)], rowsB, semrB).wait()
        off_last = pl.multiple_of(
            base + np.int32((n_chunks - 1) * _CHUNK), _CHUNK)
        pltpu.async_copy(rowsB, out_h.at[pl.ds(off_last, _CHUNK)], semoB)
        pltpu.make_async_copy(rowsA, out_h.at[pl.ds(0, _CHUNK)], semoA).wait()
        pltpu.make_async_copy(rowsB, out_h.at[pl.ds(0, _CHUNK)], semoB).wait()

    return k


def kernel(query_pts, features, buffer_voxel_index):
    n = query_pts.shape[0]
    nv, d = features.shape
    qt = query_pts.T
    qx, qy, qz = qt[0], qt[1], qt[2]
    buf = buffer_voxel_index.astype(jnp.int32)
    feat_ext = jnp.concatenate(
        [features.astype(jnp.float32), jnp.zeros((1, d), jnp.float32)], axis=0)
    return _make_kernel(n, d, nv)(qx, qy, qz, buf, feat_ext)
